# Initial kernel scaffold; baseline (speedup 1.0000x reference)
import jax
import jax.numpy as jnp
from jax.experimental import pallas as pl


def kernel(x, W, b, health_gate):
    raise NotImplementedError("write your pallas kernel here")


# speedup vs baseline: 2.2772x; 2.2772x over previous
import jax
import jax.numpy as jnp
from jax.experimental import pallas as pl

N_NODES = 16
K_SPARSE = 5


def _fused_kernel(x_ref, w_ref, b_ref, g_ref, o_ref):
    x = x_ref[...]
    w = w_ref[...]
    acts = jax.lax.dot_general(
        x, w, (((1,), (1,)), ((), ())), preferred_element_type=jnp.float32
    )
    acts = jnp.maximum(acts + b_ref[...], 0.0) * jax.nn.sigmoid(g_ref[...])

    bits = jax.lax.bitcast_convert_type(acts, jnp.int32)
    col = jax.lax.broadcasted_iota(jnp.int32, acts.shape, 1)
    key = jnp.bitwise_or(jnp.bitwise_and(bits, jnp.int32(~0xF)),
                         jnp.int32(N_NODES - 1) - col)
    work = key
    thr = None
    for _ in range(K_SPARSE):
        thr = jnp.max(work, axis=1, keepdims=True)
        work = jnp.where(work == thr, jnp.int32(-(2**31)), work)
    o_ref[...] = jnp.where(key >= thr, acts, 0.0)


def kernel(x, W, b, health_gate):
    B = x.shape[0]
    return pl.pallas_call(
        _fused_kernel,
        out_shape=jax.ShapeDtypeStruct((B, N_NODES), jnp.float32),
    )(x, W, b.reshape(1, N_NODES), health_gate.reshape(1, N_NODES))
